# baseline (speedup 1.0000x reference)
import functools

import numpy as np
import jax
import jax.numpy as jnp
from jax import lax
from jax.experimental import pallas as pl
from jax.experimental.pallas import tpu as pltpu
from jax.experimental.pallas import tpu_sc as plsc

_VOCAB = 1000000
_EMB = 64
_EH = 32
_SEQ = 200
_BATCH = 1024

_NC = 2
_NS = 16
_L = 16
_NW = _NC * _NS

_P = 512000
_MB = 12800

_NUNITS = (_SEQ // 8) * 8 * 8
_UPW = _NUNITS // _NW


def _pos_encoding():
    pos = np.arange(_SEQ, dtype=np.float32)[:, None]
    i = np.arange(0, _EMB, 2, dtype=np.float32)
    div = np.power(10000.0, i / _EMB)
    pe = np.zeros((_SEQ, _EMB), np.float32)
    pe[:, 0::2] = np.sin(pos / div)
    pe[:, 1::2] = np.cos(pos / div)
    return pe


_PE = _pos_encoding()

_N_IN_BLOCKS = _VOCAB // _MB


def _pack_body(in0_ref, in1_ref, o_ref):
    i0 = lax.broadcasted_iota(jnp.int32, (2 * _EH, 2 * _EH), 0)
    i1 = lax.broadcasted_iota(jnp.int32, (2 * _EH, 2 * _EH), 1)
    ident = (i0 == i1).astype(jnp.float32)
    dn = (((0,), (0,)), ((), ()))
    lhs = jnp.concatenate([in0_ref[...], in1_ref[...]], axis=0)
    o_ref[...] = lax.dot_general(
        lhs, ident, dn, preferred_element_type=jnp.float32)


def _mk_pack(h):
    return pl.pallas_call(
        _pack_body,
        grid=(_P // _MB,),
        in_specs=[
            pl.BlockSpec((_EH, _MB), lambda q: (h, q)),
            pl.BlockSpec(
                (_EH, _MB),
                lambda q: (h, jnp.minimum(q + _P // _MB, _N_IN_BLOCKS))),
        ],
        out_specs=pl.BlockSpec((_MB, 2 * _EH), lambda q: (q, 0)),
        out_shape=jax.ShapeDtypeStruct((_P, 2 * _EH), jnp.float32),
    )


_pack_a = _mk_pack(0)
_pack_b = _mk_pack(1)

_mesh = plsc.VectorSubcoreMesh(core_axis_name="c", subcore_axis_name="s")

_OUT_T = jax.ShapeDtypeStruct((_SEQ, 8, 8, 8, 128), jnp.float32)
_SCRATCH = [
    pltpu.VMEM((_UPW, 128), jnp.int32),
    pltpu.VMEM((_SEQ, _EMB), jnp.float32),
    pltpu.VMEM((128, _EH), jnp.float32),
    pltpu.VMEM((128, _EH), jnp.float32),
    pltpu.VMEM((4, 8, 129), jnp.float32),
    pltpu.VMEM((4, 8, 129), jnp.float32),
    pltpu.SemaphoreType.DMA,
    pltpu.SemaphoreType.DMA,
    pltpu.SemaphoreType.DMA,
    pltpu.SemaphoreType.DMA,
]


def _emb_body(je_base, x_hbm, table_hbm, pe_hbm, out_hbm,
              idx_v, pe_v, gbuf0, gbuf1, tbuf0, tbuf1,
              gsem0, gsem1, osem0, osem1, csem=None, prev_hbm=None):
    wid = lax.axis_index("s") * _NC + lax.axis_index("c")
    ubase = wid * _UPW
    eo = je_base * 8

    if prev_hbm is not None:
        def half_cp():
            return pltpu.make_async_copy(
                prev_hbm.at[pl.ds(wid * 8, 8), pl.ds(0, 4)],
                out_hbm.at[pl.ds(wid * 8, 8), pl.ds(0, 4)], csem)

        @pl.when(wid < _SEQ // 8)
        def _():
            half_cp().start()

    pltpu.sync_copy(x_hbm.at[pl.ds(ubase, _UPW)], idx_v)
    pltpu.sync_copy(pe_hbm, pe_v)

    def remap_row(k, carry):
        for c in range(128 // _L):
            v = idx_v[k, pl.ds(c * _L, _L)]
            idx_v[k, pl.ds(c * _L, _L)] = jnp.where(
                v >= _P, 2 * (v - _P) + 1, 2 * v)
        return carry

    lax.fori_loop(0, _UPW, remap_row, 0)

    gbufs = (gbuf0, gbuf1)
    tbufs = (tbuf0, tbuf1)
    gsems = (gsem0, gsem1)
    osems = (osem0, osem1)

    lane = lax.broadcasted_iota(jnp.int32, (_L,), 0)
    e8_l = lane & 7
    je_half = lane >> 3
    jidx = [je_half + 2 * jc for jc in range(_EH // _L)]
    ones_v = jnp.full((_L,), 1, jnp.int32)
    zeros_v = jnp.zeros((_L,), jnp.int32)

    def gather_cp(i, p):
        return pltpu.make_async_copy(
            table_hbm.at[idx_v.at[i]], gbufs[p], gsems[p])

    def unit_sb(i):
        v = ubase + i
        s = (v >> 6) * 8 + (v & 7)
        bb = (v >> 3) & 7
        return s, bb

    def out_cps(i, p):
        s, bb = unit_sb(i)
        return [pltpu.make_async_copy(
            tbufs[p].at[:, :, pl.ds(0, 128)],
            out_hbm.at[s, pl.ds(je_base, 4), bb], osems[p])]

    gather_cp(0, 0).start()
    gather_cp(1, 1).start()

    def pair_body(kk, carry):
        for d in range(2):
            i = 2 * kk + d
            gbuf = gbufs[d]
            tbuf = tbufs[d]
            gather_cp(i, d).wait()

            @pl.when(i >= 2)
            def _():
                for cp in out_cps(i - 2, d):
                    cp.wait()

            s, bb = unit_sb(i)
            pvs = [pe_v[s, pl.ds(eo + jc * _L, _L)]
                   for jc in range(_EH // _L)]

            def transpose_pe(t0, t_v):
                t = 4 * t0
                for dt in range(4):
                    for jc in range(_EH // _L):
                        val = gbuf[t + dt, pl.ds(jc * _L, _L)] + pvs[jc]
                        plsc.store_scatter(
                            tbuf, [jidx[jc], e8_l, t_v], val)
                    t_v = t_v + ones_v
                return t_v

            lax.fori_loop(0, 32, transpose_pe, zeros_v)

            for cp in out_cps(i, d):
                cp.start()

            @pl.when(i + 2 < _UPW)
            def _():
                gather_cp(i + 2, d).start()
        return carry

    lax.fori_loop(0, _UPW // 2, pair_body, 0)

    for d in range(2):
        for cp in out_cps(_UPW - 2 + d, d):
            cp.wait()

    if prev_hbm is not None:
        @pl.when(wid < _SEQ // 8)
        def _():
            half_cp().wait()


_emb_a = functools.partial(
    pl.kernel,
    mesh=_mesh,
    out_type=_OUT_T,
    compiler_params=pltpu.CompilerParams(use_tc_tiling_on_sc=False,
                                         needs_layout_passes=False),
    scratch_types=_SCRATCH,
)(functools.partial(_emb_body, 0))


def _emb_body_b(x_hbm, table_hbm, pe_hbm, prev_hbm, out_hbm, *rest):
    scratch, csem = rest[:-1], rest[-1]
    return _emb_body(4, x_hbm, table_hbm, pe_hbm, out_hbm, *scratch,
                     csem=csem, prev_hbm=prev_hbm)


_emb_b = functools.partial(
    pl.kernel,
    mesh=_mesh,
    out_type=_OUT_T,
    compiler_params=pltpu.CompilerParams(use_tc_tiling_on_sc=False,
                                         needs_layout_passes=False),
    scratch_types=_SCRATCH + [pltpu.SemaphoreType.DMA],
)(_emb_body_b)


def kernel(x, table):
    xt4 = x.T.reshape(_SEQ // 8, 8, 8, 128).transpose(0, 2, 1, 3)
    xt4 = xt4.reshape(_NUNITS, 128).astype(jnp.int32)
    pe = jnp.asarray(_PE)
    tt = table.T
    packed_a = _pack_a(tt, tt)
    packed_b = _pack_b(tt, tt)
    t2v_a = packed_a.reshape(2 * _P, _EH)
    t2v_b = packed_b.reshape(2 * _P, _EH)
    out5 = _emb_a(xt4, t2v_a, pe)
    out5 = _emb_b(xt4, t2v_b, pe, out5)
    return out5.transpose(2, 4, 0, 1, 3).reshape(_BATCH, _SEQ, _EMB)


# speedup vs baseline: 4.0881x; 4.0881x over previous
import functools

import numpy as np
import jax
import jax.numpy as jnp
from jax import lax
from jax.experimental import pallas as pl
from jax.experimental.pallas import tpu as pltpu
from jax.experimental.pallas import tpu_sc as plsc

_VOCAB = 1000000
_EMB = 64
_SEQ = 200
_BATCH = 1024

_NC = 2
_NS = 16
_L = 16
_NW = _NC * _NS

_P = 512000
_MB = 12800

_NUNITS = (_SEQ // 8) * 8 * 8
_UPW = _NUNITS // _NW


def _pos_encoding():
    pos = np.arange(_SEQ, dtype=np.float32)[:, None]
    i = np.arange(0, _EMB, 2, dtype=np.float32)
    div = np.power(10000.0, i / _EMB)
    pe = np.zeros((_SEQ, _EMB), np.float32)
    pe[:, 0::2] = np.sin(pos / div)
    pe[:, 1::2] = np.cos(pos / div)
    return pe


_PE = _pos_encoding()


def _pack_body(in0_ref, in1_ref, o_ref):
    i0 = lax.broadcasted_iota(jnp.int32, (_EMB, _EMB), 0)
    i1 = lax.broadcasted_iota(jnp.int32, (_EMB, _EMB), 1)
    ident = (i0 == i1).astype(jnp.float32)
    dn = (((0,), (0,)), ((), ()))
    t0 = lax.dot_general(
        in0_ref[...], ident, dn, preferred_element_type=jnp.float32)
    t1 = lax.dot_general(
        in1_ref[...], ident, dn, preferred_element_type=jnp.float32)
    o_ref[...] = jnp.concatenate([t0, t1], axis=1)


_N_IN_BLOCKS = _VOCAB // _MB

_pack = pl.pallas_call(
    _pack_body,
    grid=(_P // _MB,),
    in_specs=[
        pl.BlockSpec((_EMB, _MB), lambda q: (0, q)),
        pl.BlockSpec((_EMB, _MB),
                     lambda q: (0, jnp.minimum(q + _P // _MB, _N_IN_BLOCKS))),
    ],
    out_specs=pl.BlockSpec((_MB, 2 * _EMB), lambda q: (q, 0)),
    out_shape=jax.ShapeDtypeStruct((_P, 2 * _EMB), jnp.float32),
)

_mesh = plsc.VectorSubcoreMesh(core_axis_name="c", subcore_axis_name="s")


@functools.partial(
    pl.kernel,
    mesh=_mesh,
    out_type=jax.ShapeDtypeStruct((_SEQ, 8, 8, 8, 128), jnp.float32),
    compiler_params=pltpu.CompilerParams(use_tc_tiling_on_sc=False,
                                         needs_layout_passes=False),
    scratch_types=[
        pltpu.VMEM((_UPW, 128), jnp.int32),
        pltpu.VMEM((_SEQ, _EMB), jnp.float32),
        pltpu.VMEM((128, _EMB), jnp.float32),
        pltpu.VMEM((128, _EMB), jnp.float32),
        pltpu.VMEM((8, 8, 129), jnp.float32),
        pltpu.VMEM((8, 8, 129), jnp.float32),
        pltpu.SemaphoreType.DMA,
        pltpu.SemaphoreType.DMA,
        pltpu.SemaphoreType.DMA,
        pltpu.SemaphoreType.DMA,
    ],
)
def _emb_kernel(x_hbm, table_hbm, pe_hbm, out_hbm,
                idx_v, pe_v, gbuf0, gbuf1, tbuf0, tbuf1,
                gsem0, gsem1, osem0, osem1):
    wid = lax.axis_index("s") * _NC + lax.axis_index("c")
    ubase = wid * _UPW

    pltpu.sync_copy(x_hbm.at[pl.ds(ubase, _UPW)], idx_v)
    pltpu.sync_copy(pe_hbm, pe_v)

    def remap_row(k, carry):
        for c in range(128 // _L):
            v = idx_v[k, pl.ds(c * _L, _L)]
            idx_v[k, pl.ds(c * _L, _L)] = jnp.where(
                v >= _P, 2 * (v - _P) + 1, 2 * v)
        return carry

    lax.fori_loop(0, _UPW, remap_row, 0)

    gbufs = (gbuf0, gbuf1)
    tbufs = (tbuf0, tbuf1)
    gsems = (gsem0, gsem1)
    osems = (osem0, osem1)

    lane = lax.broadcasted_iota(jnp.int32, (_L,), 0)
    e8_l = lane & 7
    je_half = lane >> 3
    jidx = [je_half + 2 * jc for jc in range(_EMB // _L)]
    ones_v = jnp.full((_L,), 1, jnp.int32)
    zeros_v = jnp.zeros((_L,), jnp.int32)

    def gather_cp(i, p):
        return pltpu.make_async_copy(
            table_hbm.at[idx_v.at[i]], gbufs[p], gsems[p])

    def unit_sb(i):
        v = ubase + i
        s = (v >> 6) * 8 + (v & 7)
        bb = (v >> 3) & 7
        return s, bb

    def out_cps(i, p):
        s, bb = unit_sb(i)
        return [pltpu.make_async_copy(tbufs[p].at[:, :, pl.ds(0, 128)],
                                      out_hbm.at[s, :, bb], osems[p])]

    gather_cp(0, 0).start()
    gather_cp(1, 1).start()

    def pair_body(kk, carry):
        for d in range(2):
            i = 2 * kk + d
            gbuf = gbufs[d]
            tbuf = tbufs[d]
            gather_cp(i, d).wait()

            @pl.when(i >= 2)
            def _():
                for cp in out_cps(i - 2, d):
                    cp.wait()

            s, bb = unit_sb(i)
            pvs = [pe_v[s, pl.ds(jc * _L, _L)] for jc in range(_EMB // _L)]

            def transpose_pe(t0, t_v):
                t = 4 * t0
                for dt in range(4):
                    for jc in range(_EMB // _L):
                        val = gbuf[t + dt, pl.ds(jc * _L, _L)] + pvs[jc]
                        plsc.store_scatter(
                            tbuf, [jidx[jc], e8_l, t_v], val)
                    t_v = t_v + ones_v
                return t_v

            lax.fori_loop(0, 32, transpose_pe, zeros_v)

            for cp in out_cps(i, d):
                cp.start()

            @pl.when(i + 2 < _UPW)
            def _():
                gather_cp(i + 2, d).start()
        return carry

    lax.fori_loop(0, _UPW // 2, pair_body, 0)

    for d in range(2):
        for cp in out_cps(_UPW - 2 + d, d):
            cp.wait()


def kernel(x, table):
    xt4 = x.T.reshape(_SEQ // 8, 8, 8, 128).transpose(0, 2, 1, 3)
    xt4 = xt4.reshape(_NUNITS, 128).astype(jnp.int32)
    pe = jnp.asarray(_PE)
    tt = table.T
    packed = _pack(tt, tt)
    t2v = packed.reshape(2 * _P, _EMB)
    out5 = _emb_kernel(xt4, t2v, pe)
    return out5.transpose(2, 4, 0, 1, 3).reshape(_BATCH, _SEQ, _EMB)
